# Initial kernel scaffold; baseline (speedup 1.0000x reference)
import jax
import jax.numpy as jnp
from jax.experimental import pallas as pl


def kernel(images, transforms):
    raise NotImplementedError("write your pallas kernel here")


# speedup vs baseline: 758.5895x; 758.5895x over previous
import dataclasses
import functools

import jax
import jax.numpy as jnp
from jax import lax
from jax.experimental import pallas as pl
from jax.experimental.pallas import tpu as pltpu
from jax.experimental.pallas import tpu_sc as plsc

_B, _C, _H, _W = 8, 3, 512, 512
_K = 256
_N = _B * _C * _H * _W
_NLUT = _B * _C * _K
_NW = 32
_PER_W = _N // _NW
_BLOCK = 8192
_NBLK = _PER_W // _BLOCK
_PLANE_SHIFT = 18
_MAGIC_F = 8388608.0
_MAGIC_I = 0x4B000000


def _lut_body(img_hbm, lut_hbm, out_hbm, lut_v, in_v, out_v, sem):
    wid = lax.axis_index("s") * 2 + lax.axis_index("c")
    pltpu.sync_copy(lut_hbm, lut_v)
    base = wid * _PER_W

    @pl.loop(0, _NBLK)
    def _blocks(j):
        blk = base + j * _BLOCK
        pltpu.async_copy(img_hbm.at[pl.ds(blk, _BLOCK)], in_v, sem).wait()
        plane = lax.shift_right_logical(blk, _PLANE_SHIFT)
        off = plane * _K - _MAGIC_I

        @pl.loop(0, _BLOCK, step=16)
        def _vecs(i):
            v = in_v[pl.ds(i, 16)]
            r = v * 255.0 + _MAGIC_F
            idx = plsc.bitcast(r, jnp.int32) + off
            out_v[pl.ds(i, 16)] = plsc.load_gather(lut_v, [idx])

        pltpu.sync_copy(out_v, out_hbm.at[pl.ds(blk, _BLOCK)])


@jax.jit
def kernel(images, transforms):
    flat_img = images.reshape(_N)
    flat_lut = transforms.reshape(_NLUT)
    mesh = plsc.VectorSubcoreMesh(core_axis_name="c", subcore_axis_name="s")
    cp = pltpu.CompilerParams()
    if "needs_layout_passes" in pltpu.CompilerParams.__dataclass_fields__:
        cp = dataclasses.replace(cp, needs_layout_passes=False)
    run = pl.kernel(
        _lut_body,
        out_type=jax.ShapeDtypeStruct((_N,), jnp.float32),
        mesh=mesh,
        scratch_types=[
            pltpu.VMEM((_NLUT,), jnp.float32),
            pltpu.VMEM((_BLOCK,), jnp.float32),
            pltpu.VMEM((_BLOCK,), jnp.float32),
            pltpu.SemaphoreType.DMA,
        ],
        compiler_params=cp,
    )
    out = run(flat_img, flat_lut)
    return out.reshape(images.shape)


# speedup vs baseline: 791.6893x; 1.0436x over previous
import dataclasses
import functools

import jax
import jax.numpy as jnp
from jax import lax
from jax.experimental import pallas as pl
from jax.experimental.pallas import tpu as pltpu
from jax.experimental.pallas import tpu_sc as plsc

_B, _C, _H, _W = 8, 3, 512, 512
_K = 256
_N = _B * _C * _H * _W
_NLUT = _B * _C * _K
_NW = 32
_PER_W = _N // _NW
_BLOCK = 8192
_NBLK = _PER_W // _BLOCK
_PLANE_SHIFT = 18
_UNROLL = 8
_MAGIC_F = 8388608.0
_MAGIC_I = 0x4B000000


def _lut_body(img_hbm, lut_hbm, out_hbm, lut_v, in_v, out_v, sem):
    wid = lax.axis_index("s") * 2 + lax.axis_index("c")
    pltpu.sync_copy(lut_hbm, lut_v)
    base = wid * _PER_W

    @pl.loop(0, _NBLK)
    def _blocks(j):
        blk = base + j * _BLOCK
        pltpu.async_copy(img_hbm.at[pl.ds(blk, _BLOCK)], in_v, sem).wait()
        plane = lax.shift_right_logical(blk, _PLANE_SHIFT)
        off = plane * _K - _MAGIC_I

        @pl.loop(0, _BLOCK, step=16 * _UNROLL)
        def _vecs(i):
            for u in range(_UNROLL):
                s = i + 16 * u
                v = in_v[pl.ds(s, 16)]
                r = v * 255.0 + _MAGIC_F
                idx = plsc.bitcast(r, jnp.int32) + off
                out_v[pl.ds(s, 16)] = plsc.load_gather(lut_v, [idx])

        pltpu.sync_copy(out_v, out_hbm.at[pl.ds(blk, _BLOCK)])


@jax.jit
def kernel(images, transforms):
    flat_img = images.reshape(_N)
    flat_lut = transforms.reshape(_NLUT)
    mesh = plsc.VectorSubcoreMesh(core_axis_name="c", subcore_axis_name="s")
    cp = pltpu.CompilerParams()
    if "needs_layout_passes" in pltpu.CompilerParams.__dataclass_fields__:
        cp = dataclasses.replace(cp, needs_layout_passes=False)
    run = pl.kernel(
        _lut_body,
        out_type=jax.ShapeDtypeStruct((_N,), jnp.float32),
        mesh=mesh,
        scratch_types=[
            pltpu.VMEM((_NLUT,), jnp.float32),
            pltpu.VMEM((_BLOCK,), jnp.float32),
            pltpu.VMEM((_BLOCK,), jnp.float32),
            pltpu.SemaphoreType.DMA,
        ],
        compiler_params=cp,
    )
    out = run(flat_img, flat_lut)
    return out.reshape(images.shape)


# speedup vs baseline: 1016.2254x; 1.2836x over previous
import dataclasses
import functools

import jax
import jax.numpy as jnp
from jax import lax
from jax.experimental import pallas as pl
from jax.experimental.pallas import tpu as pltpu
from jax.experimental.pallas import tpu_sc as plsc

_B, _C, _H, _W = 8, 3, 512, 512
_K = 256
_N = _B * _C * _H * _W
_NLUT = _B * _C * _K
_NW = 32
_PER_W = _N // _NW
_BLOCK = 8192
_NBLK = _PER_W // _BLOCK
_PLANE_SHIFT = 18
_UNROLL = 8
_MAGIC_F = 8388608.0
_MAGIC_I = 0x4B000000


def _lut_body(img_hbm, lut_hbm, out_hbm, lut_v, in_v, out_v, sem):
    wid = lax.axis_index("s") * 2 + lax.axis_index("c")
    pltpu.sync_copy(lut_hbm, lut_v)
    base = wid * _PER_W

    @pl.loop(0, _NBLK)
    def _blocks(j):
        blk = base + j * _BLOCK
        pltpu.async_copy(img_hbm.at[pl.ds(blk, _BLOCK)], in_v, sem).wait()
        plane = lax.shift_right_logical(blk, _PLANE_SHIFT)
        off = plane * _K - _MAGIC_I

        @plsc.parallel_loop(0, _BLOCK, step=16, unroll=_UNROLL)
        def _vecs(i):
            v = in_v[pl.ds(i, 16)]
            r = v * 255.0 + _MAGIC_F
            idx = plsc.bitcast(r, jnp.int32) + off
            out_v[pl.ds(i, 16)] = plsc.load_gather(lut_v, [idx])

        pltpu.sync_copy(out_v, out_hbm.at[pl.ds(blk, _BLOCK)])


@jax.jit
def kernel(images, transforms):
    flat_img = images.reshape(_N)
    flat_lut = transforms.reshape(_NLUT)
    mesh = plsc.VectorSubcoreMesh(core_axis_name="c", subcore_axis_name="s")
    cp = pltpu.CompilerParams()
    if "needs_layout_passes" in pltpu.CompilerParams.__dataclass_fields__:
        cp = dataclasses.replace(cp, needs_layout_passes=False)
    run = pl.kernel(
        _lut_body,
        out_type=jax.ShapeDtypeStruct((_N,), jnp.float32),
        mesh=mesh,
        scratch_types=[
            pltpu.VMEM((_NLUT,), jnp.float32),
            pltpu.VMEM((_BLOCK,), jnp.float32),
            pltpu.VMEM((_BLOCK,), jnp.float32),
            pltpu.SemaphoreType.DMA,
        ],
        compiler_params=cp,
    )
    out = run(flat_img, flat_lut)
    return out.reshape(images.shape)


# speedup vs baseline: 1645.4664x; 1.6192x over previous
import dataclasses
import functools

import jax
import jax.numpy as jnp
from jax import lax
from jax.experimental import pallas as pl
from jax.experimental.pallas import tpu as pltpu
from jax.experimental.pallas import tpu_sc as plsc

_B, _C, _H, _W = 8, 3, 512, 512
_K = 256
_N = _B * _C * _H * _W
_NLUT = _B * _C * _K
_NW = 32
_ROWS = 16
_BLOCK = _ROWS * _W
_BLK_PER_PLANE = _H // _ROWS
_NBLK = _N // _BLOCK // _NW
_UNROLL = 8
_MAGIC_F = 8388608.0
_MAGIC_I = 0x4B000000


def _lut_body(img_hbm, lut_hbm, out_hbm, lut_v, in_v, out_v, sem):
    wid = lax.axis_index("s") * 2 + lax.axis_index("c")
    pltpu.sync_copy(lut_hbm, lut_v)

    @pl.loop(0, _NBLK)
    def _blocks(k):
        gblk = wid * _NBLK + k
        plane = gblk // _BLK_PER_PLANE
        row0 = gblk * _ROWS
        pltpu.async_copy(img_hbm.at[pl.ds(row0, _ROWS)], in_v, sem).wait()
        off = plane * _K - _MAGIC_I

        @pl.loop(0, _ROWS)
        def _rows(r):
            @plsc.parallel_loop(0, _W, step=16, unroll=_UNROLL)
            def _vecs(i):
                v = in_v[r, pl.ds(i, 16)]
                rr = v * 255.0 + _MAGIC_F
                idx = plsc.bitcast(rr, jnp.int32) + off
                out_v[r, pl.ds(i, 16)] = plsc.load_gather(lut_v, [idx])

        pltpu.sync_copy(out_v, out_hbm.at[pl.ds(row0, _ROWS)])


@jax.jit
def kernel(images, transforms):
    img2d = images.reshape(_B * _C * _H, _W)
    flat_lut = transforms.reshape(_NLUT)
    mesh = plsc.VectorSubcoreMesh(core_axis_name="c", subcore_axis_name="s")
    cp = pltpu.CompilerParams()
    if "needs_layout_passes" in pltpu.CompilerParams.__dataclass_fields__:
        cp = dataclasses.replace(cp, needs_layout_passes=False)
    run = pl.kernel(
        _lut_body,
        out_type=jax.ShapeDtypeStruct((_B * _C * _H, _W), jnp.float32),
        mesh=mesh,
        scratch_types=[
            pltpu.VMEM((_NLUT,), jnp.float32),
            pltpu.VMEM((_ROWS, _W), jnp.float32),
            pltpu.VMEM((_ROWS, _W), jnp.float32),
            pltpu.SemaphoreType.DMA,
        ],
        compiler_params=cp,
    )
    return run(img2d, flat_lut).reshape(_B, _C, _H, _W)
